# baseline (device time: 35157 ns/iter reference)
import jax
import jax.numpy as jnp
from jax import lax
from jax.experimental import pallas as pl
from jax.experimental.pallas import tpu as pltpu


def kernel(O, Wo):
    b, s, h, d = O.shape
    k = h * d
    n = Wo.shape[1]
    s_half = s // 2

    def body(o_hbm, w_hbm, out_ref, o_ref, w_ref, comm_ref,
             load_sems, send_sem, recv_sem):
        my_x = lax.axis_index("x")
        my_y = lax.axis_index("y")
        nbr_x = 1 - my_x

        o_load = pltpu.make_async_copy(o_hbm, o_ref, load_sems.at[0])
        w_load = pltpu.make_async_copy(w_hbm, w_ref, load_sems.at[1])
        o_load.start()
        w_load.start()

        barrier = pltpu.get_barrier_semaphore()
        pl.semaphore_signal(
            barrier, inc=1,
            device_id=(nbr_x, my_y), device_id_type=pl.DeviceIdType.MESH,
        )
        pl.semaphore_wait(barrier, 1)

        o_load.wait()
        w_load.wait()
        w = w_ref[...].astype(jnp.bfloat16)

        def partial_for(bi, s_start):
            o_blk = o_ref[bi, pl.ds(s_start, s_half), :, :]
            o_blk = o_blk.reshape(s_half, k).astype(jnp.bfloat16)
            return lax.dot_general(
                o_blk, w, (((1,), (0,)), ((), ())),
                preferred_element_type=jnp.float32,
            )

        rdmas = []
        for bi in range(b):
            comm_ref[0, bi] = partial_for(bi, nbr_x * s_half).astype(jnp.bfloat16)
            r = pltpu.make_async_remote_copy(
                src_ref=comm_ref.at[0, bi],
                dst_ref=comm_ref.at[1, bi],
                send_sem=send_sem.at[bi],
                recv_sem=recv_sem.at[bi],
                device_id=(nbr_x, my_y),
                device_id_type=pl.DeviceIdType.MESH,
            )
            r.start()
            rdmas.append(r)

        for bi in range(b):
            p_mine = partial_for(bi, my_x * s_half)
            rdmas[bi].wait()
            out_ref[bi] = p_mine + comm_ref[1, bi].astype(jnp.float32)

    return pl.pallas_call(
        body,
        out_shape=jax.ShapeDtypeStruct((b, s_half, n), jnp.float32),
        in_specs=[
            pl.BlockSpec(memory_space=pl.ANY),
            pl.BlockSpec(memory_space=pl.ANY),
        ],
        out_specs=pl.BlockSpec(memory_space=pltpu.VMEM),
        scratch_shapes=[
            pltpu.VMEM((b, s, h, d), jnp.float32),
            pltpu.VMEM((k, n), jnp.float32),
            pltpu.VMEM((2, b, s_half, n), jnp.bfloat16),
            pltpu.SemaphoreType.DMA((2,)),
            pltpu.SemaphoreType.DMA((b,)),
            pltpu.SemaphoreType.DMA((b,)),
        ],
        compiler_params=pltpu.CompilerParams(collective_id=0),
    )(O, Wo)


# device time: 33074 ns/iter; 1.0630x vs baseline; 1.0630x over previous
import jax
import jax.numpy as jnp
from jax import lax
from jax.experimental import pallas as pl
from jax.experimental.pallas import tpu as pltpu


def kernel(O, Wo):
    b, s, h, d = O.shape
    k = h * d
    n = Wo.shape[1]
    s_half = s // 2

    Ot = jnp.transpose(O, (0, 2, 3, 1))

    def body(ot_ref, w_ref, out_hbm, out_ref, comm_ref,
             send_sem, recv_sem, store_sem):
        my_x = lax.axis_index("x")
        my_y = lax.axis_index("y")
        nbr_x = 1 - my_x

        barrier = pltpu.get_barrier_semaphore()
        pl.semaphore_signal(
            barrier, inc=1,
            device_id=(nbr_x, my_y), device_id_type=pl.DeviceIdType.MESH,
        )
        pl.semaphore_wait(barrier, 1)

        w = w_ref[...].astype(jnp.bfloat16)

        def partial_for(bi, s_start):
            lhs = ot_ref[bi, :, :, pl.ds(s_start, s_half)]
            lhs = lhs.reshape(k, s_half).astype(jnp.bfloat16)
            return lax.dot_general(
                lhs, w, (((0,), (0,)), ((), ())),
                preferred_element_type=jnp.float32,
            )

        rdmas = []
        for bi in range(b):
            comm_ref[0, bi] = partial_for(bi, nbr_x * s_half).astype(jnp.bfloat16)
            r = pltpu.make_async_remote_copy(
                src_ref=comm_ref.at[0, bi],
                dst_ref=comm_ref.at[1, bi],
                send_sem=send_sem.at[bi],
                recv_sem=recv_sem.at[bi],
                device_id=(nbr_x, my_y),
                device_id_type=pl.DeviceIdType.MESH,
            )
            r.start()
            rdmas.append(r)

        stores = []
        for bi in range(b):
            p_mine = partial_for(bi, my_x * s_half)
            rdmas[bi].wait()
            out_ref[bi] = p_mine + comm_ref[1, bi].astype(jnp.float32)
            st = pltpu.make_async_copy(
                out_ref.at[bi], out_hbm.at[bi], store_sem.at[bi]
            )
            st.start()
            stores.append(st)
        for st in stores:
            st.wait()

    return pl.pallas_call(
        body,
        out_shape=jax.ShapeDtypeStruct((b, s_half, n), jnp.float32),
        in_specs=[
            pl.BlockSpec(memory_space=pltpu.VMEM),
            pl.BlockSpec(memory_space=pltpu.VMEM),
        ],
        out_specs=pl.BlockSpec(memory_space=pl.ANY),
        scratch_shapes=[
            pltpu.VMEM((b, s_half, n), jnp.float32),
            pltpu.VMEM((2, b, s_half, n), jnp.bfloat16),
            pltpu.SemaphoreType.DMA((b,)),
            pltpu.SemaphoreType.DMA((b,)),
            pltpu.SemaphoreType.DMA((b,)),
        ],
        compiler_params=pltpu.CompilerParams(collective_id=0),
    )(Ot, Wo)


# device time: 27190 ns/iter; 1.2930x vs baseline; 1.2164x over previous
import jax
import jax.numpy as jnp
from jax import lax
from jax.experimental import pallas as pl
from jax.experimental.pallas import tpu as pltpu


def kernel(O, Wo):
    b, s, h, d = O.shape
    k = h * d
    n = Wo.shape[1]
    s_half = s // 2
    n_half = n // 2

    Ot = jnp.transpose(O, (0, 2, 3, 1))

    def body(ot_hbm, w_hbm, out_hbm,
             ot_ref, w_ref, out_ref, send_ref, xrecv_ref, yrecv_ref,
             load_sems, dsend_sems, xrecv_sems, fsend_sems, yrecv_sems,
             store_sems):
        my_x = lax.axis_index("x")
        my_y = lax.axis_index("y")
        nbr_x = 1 - my_x
        nbr_y = 1 - my_y

        w_load = pltpu.make_async_copy(w_hbm, w_ref, load_sems.at[0])
        w_load.start()
        o_loads = []
        for bi in range(b):
            ld = pltpu.make_async_copy(
                ot_hbm.at[bi], ot_ref.at[bi], load_sems.at[1 + bi]
            )
            ld.start()
            o_loads.append(ld)

        barrier = pltpu.get_barrier_semaphore()
        for dev in [(nbr_x, my_y), (my_x, nbr_y)]:
            pl.semaphore_signal(
                barrier, inc=1,
                device_id=dev, device_id_type=pl.DeviceIdType.MESH,
            )
        pl.semaphore_wait(barrier, 2)

        w_load.wait()
        w_bf = w_ref[...].astype(jnp.bfloat16)
        w_half = w_ref[:, pl.ds(my_y * n_half, n_half)].astype(jnp.bfloat16)

        def lhs_for(bi, s_start):
            blk = ot_ref[bi, :, :, pl.ds(s_start, s_half)]
            return blk.reshape(k, s_half).astype(jnp.bfloat16)

        directs = []
        for bi in range(b):
            o_loads[bi].wait()
            p = lax.dot_general(
                lhs_for(bi, nbr_x * s_half), w_half,
                (((0,), (0,)), ((), ())),
                preferred_element_type=jnp.float32,
            )
            send_ref[bi] = p.astype(jnp.bfloat16)
            r = pltpu.make_async_remote_copy(
                src_ref=send_ref.at[bi],
                dst_ref=xrecv_ref.at[bi],
                send_sem=dsend_sems.at[bi],
                recv_sem=xrecv_sems.at[bi],
                device_id=(nbr_x, my_y),
                device_id_type=pl.DeviceIdType.MESH,
            )
            r.start()
            directs.append(r)

        forwards = []
        for bi in range(b):
            out_ref[bi] = lax.dot_general(
                lhs_for(bi, my_x * s_half), w_bf,
                (((0,), (0,)), ((), ())),
                preferred_element_type=jnp.float32,
            )
            directs[bi].wait()
            f = pltpu.make_async_remote_copy(
                src_ref=xrecv_ref.at[bi],
                dst_ref=yrecv_ref.at[bi],
                send_sem=fsend_sems.at[bi],
                recv_sem=yrecv_sems.at[bi],
                device_id=(my_x, nbr_y),
                device_id_type=pl.DeviceIdType.MESH,
            )
            f.start()
            forwards.append(f)
            out_ref[bi, :, pl.ds(my_y * n_half, n_half)] = (
                out_ref[bi, :, pl.ds(my_y * n_half, n_half)]
                + xrecv_ref[bi].astype(jnp.float32)
            )

        stores = []
        for bi in range(b):
            forwards[bi].wait()
            out_ref[bi, :, pl.ds(nbr_y * n_half, n_half)] = (
                out_ref[bi, :, pl.ds(nbr_y * n_half, n_half)]
                + yrecv_ref[bi].astype(jnp.float32)
            )
            st = pltpu.make_async_copy(
                out_ref.at[bi], out_hbm.at[bi], store_sems.at[bi]
            )
            st.start()
            stores.append(st)
        for st in stores:
            st.wait()

    return pl.pallas_call(
        body,
        out_shape=jax.ShapeDtypeStruct((b, s_half, n), jnp.float32),
        in_specs=[
            pl.BlockSpec(memory_space=pl.ANY),
            pl.BlockSpec(memory_space=pl.ANY),
        ],
        out_specs=pl.BlockSpec(memory_space=pl.ANY),
        scratch_shapes=[
            pltpu.VMEM((b, h, d, s), jnp.float32),
            pltpu.VMEM((k, n), jnp.float32),
            pltpu.VMEM((b, s_half, n), jnp.float32),
            pltpu.VMEM((b, s_half, n_half), jnp.bfloat16),
            pltpu.VMEM((b, s_half, n_half), jnp.bfloat16),
            pltpu.VMEM((b, s_half, n_half), jnp.bfloat16),
            pltpu.SemaphoreType.DMA((1 + b,)),
            pltpu.SemaphoreType.DMA((b,)),
            pltpu.SemaphoreType.DMA((b,)),
            pltpu.SemaphoreType.DMA((b,)),
            pltpu.SemaphoreType.DMA((b,)),
            pltpu.SemaphoreType.DMA((b,)),
        ],
        compiler_params=pltpu.CompilerParams(collective_id=0),
    )(Ot, Wo)


# device time: 24515 ns/iter; 1.4341x vs baseline; 1.1091x over previous
import jax
import jax.numpy as jnp
from jax import lax
from jax.experimental import pallas as pl
from jax.experimental.pallas import tpu as pltpu


def kernel(O, Wo):
    b, s, h, d = O.shape
    k = h * d
    n = Wo.shape[1]
    s_half = s // 2
    n_half = n // 2

    Ot = jnp.transpose(O, (0, 2, 3, 1))

    def body(ot_hbm, w_hbm, out_hbm,
             ot_ref, w_ref, out_ref, send_ref, xrecv_ref, yrecv_ref,
             load_sems, dsend_sems, xrecv_sems, fsend_sems, yrecv_sems,
             store_sems):
        my_x = lax.axis_index("x")
        my_y = lax.axis_index("y")
        nbr_x = 1 - my_x
        nbr_y = 1 - my_y

        w_load = pltpu.make_async_copy(w_hbm, w_ref, load_sems.at[0])
        w_load.start()
        o_loads = []
        for bi in range(b):
            ld = pltpu.make_async_copy(
                ot_hbm.at[bi], ot_ref.at[bi], load_sems.at[1 + bi]
            )
            ld.start()
            o_loads.append(ld)

        barrier = pltpu.get_barrier_semaphore()
        for dev in [(nbr_x, my_y), (my_x, nbr_y)]:
            pl.semaphore_signal(
                barrier, inc=1,
                device_id=dev, device_id_type=pl.DeviceIdType.MESH,
            )
        pl.semaphore_wait(barrier, 2)

        w_load.wait()
        w_bf = w_ref[...].astype(jnp.bfloat16)
        w_half = w_ref[:, pl.ds(my_y * n_half, n_half)].astype(jnp.bfloat16)

        def lhs_for(bi, s_start):
            blk = ot_ref[bi, :, :, pl.ds(s_start, s_half)]
            return blk.reshape(k, s_half).astype(jnp.bfloat16)

        directs = []
        for bi in range(b):
            o_loads[bi].wait()
            p = lax.dot_general(
                lhs_for(bi, nbr_x * s_half), w_half,
                (((0,), (0,)), ((), ())),
                preferred_element_type=jnp.float32,
            )
            send_ref[bi] = p.astype(jnp.bfloat16)
            r = pltpu.make_async_remote_copy(
                src_ref=send_ref.at[bi],
                dst_ref=xrecv_ref.at[bi],
                send_sem=dsend_sems.at[bi],
                recv_sem=xrecv_sems.at[bi],
                device_id=(nbr_x, my_y),
                device_id_type=pl.DeviceIdType.MESH,
            )
            r.start()
            directs.append(r)

        forwards = []
        for bi in range(b):
            out_ref[bi] = lax.dot_general(
                lhs_for(bi, my_x * s_half), w_bf,
                (((0,), (0,)), ((), ())),
                preferred_element_type=jnp.float32,
            )
            directs[bi].wait()
            f = pltpu.make_async_remote_copy(
                src_ref=xrecv_ref.at[bi],
                dst_ref=yrecv_ref.at[bi],
                send_sem=fsend_sems.at[bi],
                recv_sem=yrecv_sems.at[bi],
                device_id=(my_x, nbr_y),
                device_id_type=pl.DeviceIdType.MESH,
            )
            f.start()
            forwards.append(f)
            out_ref[bi, :, pl.ds(my_y * n_half, n_half)] = (
                out_ref[bi, :, pl.ds(my_y * n_half, n_half)]
                + xrecv_ref[bi].astype(jnp.float32)
            )

        stores = []
        for bi in range(b):
            forwards[bi].wait()
            out_ref[bi, :, pl.ds(nbr_y * n_half, n_half)] = (
                out_ref[bi, :, pl.ds(nbr_y * n_half, n_half)]
                + yrecv_ref[bi].astype(jnp.float32)
            )
            st = pltpu.make_async_copy(
                out_ref.at[bi], out_hbm.at[bi], store_sems.at[bi]
            )
            st.start()
            stores.append(st)
        for st in stores:
            st.wait()

    return pl.pallas_call(
        body,
        out_shape=jax.ShapeDtypeStruct((b, s_half, n), jnp.float32),
        in_specs=[
            pl.BlockSpec(memory_space=pltpu.MemorySpace.HBM),
            pl.BlockSpec(memory_space=pltpu.MemorySpace.HBM),
        ],
        out_specs=pl.BlockSpec(memory_space=pl.ANY),
        scratch_shapes=[
            pltpu.VMEM((b, h, d, s), jnp.float32),
            pltpu.VMEM((k, n), jnp.float32),
            pltpu.VMEM((b, s_half, n), jnp.float32),
            pltpu.VMEM((b, s_half, n_half), jnp.bfloat16),
            pltpu.VMEM((b, s_half, n_half), jnp.bfloat16),
            pltpu.VMEM((b, s_half, n_half), jnp.bfloat16),
            pltpu.SemaphoreType.DMA((1 + b,)),
            pltpu.SemaphoreType.DMA((b,)),
            pltpu.SemaphoreType.DMA((b,)),
            pltpu.SemaphoreType.DMA((b,)),
            pltpu.SemaphoreType.DMA((b,)),
            pltpu.SemaphoreType.DMA((b,)),
        ],
        compiler_params=pltpu.CompilerParams(collective_id=0),
    )(
        pltpu.with_memory_space_constraint(Ot, pltpu.MemorySpace.HBM),
        pltpu.with_memory_space_constraint(Wo, pltpu.MemorySpace.HBM),
    )
